# Initial kernel scaffold; baseline (speedup 1.0000x reference)
import jax
import jax.numpy as jnp
from jax.experimental import pallas as pl


def kernel(x, pe_table):
    raise NotImplementedError("write your pallas kernel here")


# speedup vs baseline: 1.9081x; 1.9081x over previous
import jax
import jax.numpy as jnp
from jax.experimental import pallas as pl


_BS = 512


def _pe_add_kernel(x_ref, pe_ref, o_ref):
    o_ref[...] = x_ref[...] + pe_ref[...][None]


def kernel(x, pe_table):
    B, S, F = x.shape
    grid = (S // _BS, B)
    return pl.pallas_call(
        _pe_add_kernel,
        grid=grid,
        in_specs=[
            pl.BlockSpec((1, _BS, F), lambda s, b: (b, s, 0)),
            pl.BlockSpec((_BS, F), lambda s, b: (s, 0)),
        ],
        out_specs=pl.BlockSpec((1, _BS, F), lambda s, b: (b, s, 0)),
        out_shape=jax.ShapeDtypeStruct((B, S, F), x.dtype),
    )(x, pe_table)


# speedup vs baseline: 2.1127x; 1.1072x over previous
import jax
import jax.numpy as jnp
from jax.experimental import pallas as pl


_BS = 1024


def _pe_add_kernel(x_ref, pe_ref, o_ref):
    o_ref[...] = x_ref[...] + pe_ref[...][None]


def kernel(x, pe_table):
    B, S, F = x.shape
    grid = (S // _BS, B)
    return pl.pallas_call(
        _pe_add_kernel,
        grid=grid,
        in_specs=[
            pl.BlockSpec((1, _BS, F), lambda s, b: (b, s, 0)),
            pl.BlockSpec((_BS, F), lambda s, b: (s, 0)),
        ],
        out_specs=pl.BlockSpec((1, _BS, F), lambda s, b: (b, s, 0)),
        out_shape=jax.ShapeDtypeStruct((B, S, F), x.dtype),
    )(x, pe_table)


# speedup vs baseline: 2.2477x; 1.0639x over previous
import jax
import jax.numpy as jnp
from jax.experimental import pallas as pl


_BS = 2048


def _pe_add_kernel(x_ref, pe_ref, o_ref):
    o_ref[...] = x_ref[...] + pe_ref[...][None]


def kernel(x, pe_table):
    B, S, F = x.shape
    grid = (S // _BS, B)
    return pl.pallas_call(
        _pe_add_kernel,
        grid=grid,
        in_specs=[
            pl.BlockSpec((1, _BS, F), lambda s, b: (b, s, 0)),
            pl.BlockSpec((_BS, F), lambda s, b: (s, 0)),
        ],
        out_specs=pl.BlockSpec((1, _BS, F), lambda s, b: (b, s, 0)),
        out_shape=jax.ShapeDtypeStruct((B, S, F), x.dtype),
    )(x, pe_table)


# speedup vs baseline: 2.2559x; 1.0037x over previous
import jax
import jax.numpy as jnp
from jax.experimental import pallas as pl
from jax.experimental.pallas import tpu as pltpu


_BS = 2048


def _pe_add_kernel(x_ref, pe_ref, o_ref):
    o_ref[...] = x_ref[...] + pe_ref[...][None]


def kernel(x, pe_table):
    B, S, F = x.shape
    grid = (S // _BS, B)
    return pl.pallas_call(
        _pe_add_kernel,
        grid=grid,
        in_specs=[
            pl.BlockSpec((1, _BS, F), lambda s, b: (b, s, 0)),
            pl.BlockSpec((_BS, F), lambda s, b: (s, 0)),
        ],
        out_specs=pl.BlockSpec((1, _BS, F), lambda s, b: (b, s, 0)),
        out_shape=jax.ShapeDtypeStruct((B, S, F), x.dtype),
        compiler_params=pltpu.CompilerParams(
            dimension_semantics=("parallel", "parallel"),
        ),
    )(x, pe_table)
